# Initial kernel scaffold; baseline (speedup 1.0000x reference)
import jax
import jax.numpy as jnp
from jax.experimental import pallas as pl


def kernel(x, edge_index, W1, b1, W2, b2, W3, b3, Wc, bc):
    raise NotImplementedError("write your pallas kernel here")


# speedup vs baseline: 10.8415x; 10.8415x over previous
import functools

import jax
import jax.numpy as jnp
from jax import lax
from jax.experimental import pallas as pl
from jax.experimental.pallas import tpu as pltpu
from jax.experimental.pallas import tpu_sc as plsc

N_NODES = 10000
N_EDGES = 320000
NC = 2
NS = 16
NW = NC * NS
CHUNK = 128
K_CHUNKS = 79
E_PAD = NW * K_CHUNKS * CHUNK
ACC_ROWS = 10240
STRIPE = ACC_ROWS // NS


def _make_sc_pass(feat, gather):
  mesh = plsc.VectorSubcoreMesh(core_axis_name="c", subcore_axis_name="s")

  scratch = []
  if gather:
    scratch.append(pltpu.VMEM((K_CHUNKS, CHUNK), jnp.int32))
  scratch += [
      pltpu.VMEM((K_CHUNKS, CHUNK), jnp.int32),
      pltpu.VMEM((CHUNK, feat), jnp.float32),
      pltpu.VMEM_SHARED((ACC_ROWS, feat), jnp.float32),
      pltpu.SemaphoreType.DMA,
  ]

  def body(*refs):
    if gather:
      (src_hbm, dst_hbm, t_hbm, out_hbm,
       src_v, dst_v, rows_v, acc, sem) = refs
    else:
      (dst_hbm, out_hbm, dst_v, rows_v, acc, sem) = refs

    c = lax.axis_index("c")
    s = lax.axis_index("s")

    def fill_rows(val):
      vec = jnp.full((16,), val, jnp.float32)
      def fb(i, carry):
        for jj in range(feat // 16):
          rows_v[i, pl.ds(jj * 16, 16)] = vec
        return carry
      lax.fori_loop(0, CHUNK, fb, 0)

    fill_rows(0.0)
    base = s * STRIPE
    for k in range(STRIPE // 64):
      pltpu.sync_copy(rows_v.at[pl.ds(0, 64)],
                      acc.at[pl.ds(base + k * 64, 64)])
    plsc.subcore_barrier()

    if not gather:
      fill_rows(1.0)

    pltpu.sync_copy(dst_hbm.at[c, s], dst_v)
    if gather:
      pltpu.sync_copy(src_hbm.at[c, s], src_v)

    def chunk_body(j, carry):
      if gather:
        pltpu.async_copy(t_hbm.at[src_v.at[j]], rows_v, sem).wait()
      pltpu.sync_copy(rows_v, acc.at[dst_v.at[j]], add=True)
      return carry
    lax.fori_loop(0, K_CHUNKS, chunk_body, 0)

    plsc.subcore_barrier()

    for k in range(STRIPE // CHUNK):
      pltpu.sync_copy(acc.at[pl.ds(base + k * CHUNK, CHUNK)], rows_v)
      pltpu.sync_copy(rows_v, out_hbm.at[c, pl.ds(base + k * CHUNK, CHUNK)])

  return pl.kernel(
      body,
      out_type=jax.ShapeDtypeStruct((NC, ACC_ROWS, feat), jnp.float32),
      mesh=mesh,
      scratch_types=scratch,
  )


_ROWS = 1000
_GRID = N_NODES // _ROWS


def _row_spec(feat):
  return pl.BlockSpec((_ROWS, feat), lambda i: (i, 0))


def _full_spec(r, cdim):
  return pl.BlockSpec((r, cdim), lambda i: (0, 0))


def _tc_first(x, w1, d0, d1):
  def body(x_ref, w_ref, d0_ref, d1_ref, o_ref):
    dinv = lax.rsqrt(d0_ref[...] + d1_ref[...] + 1.0)
    o_ref[...] = dinv * jnp.dot(x_ref[...], w_ref[...],
                                preferred_element_type=jnp.float32)
  return pl.pallas_call(
      body,
      grid=(_GRID,),
      in_specs=[_row_spec(128), _full_spec(128, 128),
                _row_spec(1), _row_spec(1)],
      out_specs=_row_spec(128),
      out_shape=jax.ShapeDtypeStruct((N_NODES, 128), jnp.float32),
  )(x, w1, d0, d1)


def _tc_mid(p0, p1, t, d0, d1, b, w, fout):
  def body(p0_ref, p1_ref, t_ref, d0_ref, d1_ref, b_ref, w_ref, o_ref):
    dinv = lax.rsqrt(d0_ref[...] + d1_ref[...] + 1.0)
    h = jnp.tanh(dinv * (p0_ref[...] + p1_ref[...] + t_ref[...]) + b_ref[...])
    o_ref[...] = dinv * jnp.dot(h, w_ref[...],
                                preferred_element_type=jnp.float32)
  return pl.pallas_call(
      body,
      grid=(_GRID,),
      in_specs=[_row_spec(128), _row_spec(128), _row_spec(128),
                _row_spec(1), _row_spec(1),
                _full_spec(1, 128), _full_spec(128, fout)],
      out_specs=_row_spec(fout),
      out_shape=jax.ShapeDtypeStruct((N_NODES, fout), jnp.float32),
  )(p0, p1, t, d0, d1, b, w)


def _tc_last(p0, p1, t, d0, d1, b3, wc, bc):
  def body(p0_ref, p1_ref, t_ref, d0_ref, d1_ref, b_ref, wc_ref, bc_ref,
           out_ref, h_ref):
    dinv = lax.rsqrt(d0_ref[...] + d1_ref[...] + 1.0)
    acc = (p0_ref[...] + p1_ref[...] + t_ref[...])[:, :64]
    h = jnp.tanh(dinv * acc + b_ref[...])
    h_ref[...] = h
    out_ref[...] = jnp.dot(h, wc_ref[...],
                           preferred_element_type=jnp.float32) + bc_ref[...]
  return pl.pallas_call(
      body,
      grid=(_GRID,),
      in_specs=[_row_spec(128), _row_spec(128), _row_spec(128),
                _row_spec(1), _row_spec(1),
                _full_spec(1, 64), _full_spec(64, 16), _full_spec(1, 16)],
      out_specs=[_row_spec(16), _row_spec(64)],
      out_shape=[jax.ShapeDtypeStruct((N_NODES, 16), jnp.float32),
                 jax.ShapeDtypeStruct((N_NODES, 64), jnp.float32)],
  )(p0, p1, t, d0, d1, b3, wc, bc)


_sc_deg = _make_sc_pass(16, gather=False)
_sc_agg128 = _make_sc_pass(128, gather=True)


def kernel(x, edge_index, W1, b1, W2, b2, W3, b3, Wc, bc):
  ei = edge_index.astype(jnp.int32)
  npad = E_PAD - N_EDGES
  src = jnp.concatenate([ei[0], jnp.zeros((npad,), jnp.int32)])
  dst = jnp.concatenate([ei[1], jnp.full((npad,), N_NODES, jnp.int32)])
  src_r = src.reshape(NC, NS, K_CHUNKS, CHUNK)
  dst_r = dst.reshape(NC, NS, K_CHUNKS, CHUNK)

  deg_parts = _sc_deg(dst_r)
  d0 = deg_parts[0, :N_NODES, 0:1]
  d1 = deg_parts[1, :N_NODES, 0:1]

  b1r = b1.reshape(1, 128)
  b2r = b2.reshape(1, 128)
  b3r = b3.reshape(1, 64)
  bcr = bc.reshape(1, 16)

  t1 = _tc_first(x, W1, d0, d1)
  a1 = _sc_agg128(src_r, dst_r, t1)
  t2 = _tc_mid(a1[0, :N_NODES], a1[1, :N_NODES], t1, d0, d1, b1r, W2, 128)
  a2 = _sc_agg128(src_r, dst_r, t2)
  w3p = jnp.pad(W3, ((0, 0), (0, 64)))
  t3 = _tc_mid(a2[0, :N_NODES], a2[1, :N_NODES], t2, d0, d1, b2r, w3p, 128)
  a3 = _sc_agg128(src_r, dst_r, t3)
  out, h3 = _tc_last(a3[0, :N_NODES], a3[1, :N_NODES], t3, d0, d1,
                     b3r, Wc, bcr)
  return (out, h3)
